# initial kernel scaffold (unmeasured)
import jax
import jax.numpy as jnp
from jax import lax
from jax.experimental import pallas as pl
from jax.experimental.pallas import tpu as pltpu

N_DEV = 16
SQ = 256
D = 1024
HQ_PER = 8
DH = 128
SKV = 4096
SCALE = 0.08838834764831843

CH = SQ // N_DEV


def _attention_partial(x2, Wq, K2, V2, Wo):

    def body(x_ref, wq_ref, k_ref, v_ref, wo_ref, out_ref):
        h = pl.program_id(0)
        q = jnp.dot(x_ref[...], wq_ref[...], preferred_element_type=jnp.float32)
        k = k_ref[:, 0, :]
        v = v_ref[:, 0, :]
        s = lax.dot_general(
            q, k, (((1,), (1,)), ((), ())), preferred_element_type=jnp.float32
        ) * SCALE
        m = jnp.max(s, axis=1, keepdims=True)
        p = jnp.exp(s - m)
        l = jnp.sum(p, axis=1, keepdims=True)
        attn = jnp.dot(p, v, preferred_element_type=jnp.float32) / l
        contrib = jnp.dot(attn, wo_ref[...], preferred_element_type=jnp.float32)

        @pl.when(h == 0)
        def _():
            out_ref[...] = contrib

        @pl.when(h > 0)
        def _():
            out_ref[...] += contrib

    return pl.pallas_call(
        body,
        grid=(HQ_PER,),
        in_specs=[
            pl.BlockSpec((SQ, D), lambda h: (0, 0)),
            pl.BlockSpec((D, DH), lambda h: (0, h)),
            pl.BlockSpec((SKV, 1, DH), lambda h: (0, h, 0)),
            pl.BlockSpec((SKV, 1, DH), lambda h: (0, h, 0)),
            pl.BlockSpec((DH, D), lambda h: (h, 0)),
        ],
        out_specs=pl.BlockSpec((SQ, D), lambda h: (0, 0)),
        out_shape=jax.ShapeDtypeStruct((SQ, D), jnp.float32),
    )(x2, Wq, K2, V2, Wo)


def _ring_allreduce(partial):

    def body(p_ref, out_ref, r_ref, send_sems, recv_sems):
        me = lax.axis_index("i")
        left = lax.rem(me - 1 + N_DEV, N_DEV)
        right = lax.rem(me + 1, N_DEV)

        barrier_sem = pltpu.get_barrier_semaphore()
        pl.semaphore_signal(
            barrier_sem, inc=1, device_id=(left,),
            device_id_type=pl.DeviceIdType.MESH,
        )
        pl.semaphore_signal(
            barrier_sem, inc=1, device_id=(right,),
            device_id_type=pl.DeviceIdType.MESH,
        )
        pl.semaphore_wait(barrier_sem, 2)

        out_ref[0] = p_ref[...]

        for s in range(N_DEV - 1):
            c_send = lax.rem(me - s + N_DEV, N_DEV)
            rdma = pltpu.make_async_remote_copy(
                src_ref=out_ref.at[0, pl.ds(c_send * CH, CH), :],
                dst_ref=r_ref.at[s],
                send_sem=send_sems.at[s],
                recv_sem=recv_sems.at[s],
                device_id=(right,),
                device_id_type=pl.DeviceIdType.MESH,
            )
            rdma.start()
            rdma.wait()
            c_recv = lax.rem(me - s - 1 + N_DEV, N_DEV)
            rows = pl.ds(c_recv * CH, CH)
            out_ref[0, rows, :] += r_ref[s]

        for s in range(N_DEV - 1):
            g = lax.rem(me + 1 - s + N_DEV, N_DEV)
            rows = pl.ds(g * CH, CH)
            rdma = pltpu.make_async_remote_copy(
                src_ref=out_ref.at[0, rows, :],
                dst_ref=out_ref.at[0, rows, :],
                send_sem=send_sems.at[N_DEV - 1 + s],
                recv_sem=recv_sems.at[N_DEV - 1 + s],
                device_id=(right,),
                device_id_type=pl.DeviceIdType.MESH,
            )
            rdma.start()
            rdma.wait()

    return pl.pallas_call(
        body,
        out_shape=jax.ShapeDtypeStruct((1, SQ, D), jnp.float32),
        in_specs=[pl.BlockSpec(memory_space=pltpu.VMEM)],
        out_specs=pl.BlockSpec(memory_space=pltpu.VMEM),
        scratch_shapes=[
            pltpu.VMEM((N_DEV - 1, CH, D), jnp.float32),
            pltpu.SemaphoreType.DMA((2 * (N_DEV - 1),)),
            pltpu.SemaphoreType.DMA((2 * (N_DEV - 1),)),
        ],
        compiler_params=pltpu.CompilerParams(collective_id=0),
    )(partial)


def kernel(x, Wq, Wo, K_ext, V_ext):
    x2 = x.reshape(SQ, D)
    K2 = K_ext.reshape(SKV, HQ_PER, DH)
    V2 = V_ext.reshape(SKV, HQ_PER, DH)
    partial = _attention_partial(x2, Wq, K2, V2, Wo)
    return _ring_allreduce(partial)


# baseline (device time: 139288 ns/iter reference)
import jax
import jax.numpy as jnp
from jax import lax
from jax.experimental import pallas as pl
from jax.experimental.pallas import tpu as pltpu

N_DEV = 16
SQ = 256
D = 1024
HQ_PER = 8
DH = 128
SKV = 4096
SCALE = 0.08838834764831843

CH = SQ // N_DEV


def _attention_partial(x2, Wq, K2, V2, Wo):

    def body(x_ref, wq_ref, k_ref, v_ref, wo_ref, out_ref):
        h = pl.program_id(0)
        q = jnp.dot(x_ref[...], wq_ref[...], preferred_element_type=jnp.float32)
        k = k_ref[0]
        v = v_ref[0]
        s = lax.dot_general(
            q, k, (((1,), (1,)), ((), ())), preferred_element_type=jnp.float32
        ) * SCALE
        m = jnp.max(s, axis=1, keepdims=True)
        p = jnp.exp(s - m)
        l = jnp.sum(p, axis=1, keepdims=True)
        attn = jnp.dot(p, v, preferred_element_type=jnp.float32) / l
        contrib = jnp.dot(attn, wo_ref[...], preferred_element_type=jnp.float32)

        @pl.when(h == 0)
        def _():
            out_ref[...] = contrib

        @pl.when(h > 0)
        def _():
            out_ref[...] += contrib

    return pl.pallas_call(
        body,
        grid=(HQ_PER,),
        in_specs=[
            pl.BlockSpec((SQ, D), lambda h: (0, 0)),
            pl.BlockSpec((D, DH), lambda h: (0, h)),
            pl.BlockSpec((1, SKV, DH), lambda h: (h, 0, 0)),
            pl.BlockSpec((1, SKV, DH), lambda h: (h, 0, 0)),
            pl.BlockSpec((DH, D), lambda h: (h, 0)),
        ],
        out_specs=pl.BlockSpec((SQ, D), lambda h: (0, 0)),
        out_shape=jax.ShapeDtypeStruct((SQ, D), jnp.float32),
    )(x2, Wq, K2, V2, Wo)


def _ring_allreduce(partial):

    def body(p_ref, out_ref, r_ref, send_sems, recv_sems):
        me = lax.axis_index("i")
        left = lax.rem(me - 1 + N_DEV, N_DEV)
        right = lax.rem(me + 1, N_DEV)

        barrier_sem = pltpu.get_barrier_semaphore()
        pl.semaphore_signal(
            barrier_sem, inc=1, device_id=(left,),
            device_id_type=pl.DeviceIdType.MESH,
        )
        pl.semaphore_signal(
            barrier_sem, inc=1, device_id=(right,),
            device_id_type=pl.DeviceIdType.MESH,
        )
        pl.semaphore_wait(barrier_sem, 2)

        out_ref[0] = p_ref[...]

        for s in range(N_DEV - 1):
            c_send = lax.rem(me - s + N_DEV, N_DEV)
            rdma = pltpu.make_async_remote_copy(
                src_ref=out_ref.at[0, pl.ds(c_send * CH, CH), :],
                dst_ref=r_ref.at[s],
                send_sem=send_sems.at[s],
                recv_sem=recv_sems.at[s],
                device_id=(right,),
                device_id_type=pl.DeviceIdType.MESH,
            )
            rdma.start()
            rdma.wait()
            c_recv = lax.rem(me - s - 1 + N_DEV, N_DEV)
            rows = pl.ds(c_recv * CH, CH)
            out_ref[0, rows, :] += r_ref[s]

        for s in range(N_DEV - 1):
            g = lax.rem(me + 1 - s + N_DEV, N_DEV)
            rows = pl.ds(g * CH, CH)
            rdma = pltpu.make_async_remote_copy(
                src_ref=out_ref.at[0, rows, :],
                dst_ref=out_ref.at[0, rows, :],
                send_sem=send_sems.at[N_DEV - 1 + s],
                recv_sem=recv_sems.at[N_DEV - 1 + s],
                device_id=(right,),
                device_id_type=pl.DeviceIdType.MESH,
            )
            rdma.start()
            rdma.wait()

    return pl.pallas_call(
        body,
        out_shape=jax.ShapeDtypeStruct((1, SQ, D), jnp.float32),
        in_specs=[pl.BlockSpec(memory_space=pltpu.VMEM)],
        out_specs=pl.BlockSpec(memory_space=pltpu.VMEM),
        scratch_shapes=[
            pltpu.VMEM((N_DEV - 1, CH, D), jnp.float32),
            pltpu.SemaphoreType.DMA((2 * (N_DEV - 1),)),
            pltpu.SemaphoreType.DMA((2 * (N_DEV - 1),)),
        ],
        compiler_params=pltpu.CompilerParams(collective_id=0),
    )(partial)


def kernel(x, Wq, Wo, K_ext, V_ext):
    x2 = x.reshape(SQ, D)
    K2 = K_ext.reshape(SKV, HQ_PER, DH).transpose(1, 0, 2)
    V2 = V_ext.reshape(SKV, HQ_PER, DH).transpose(1, 0, 2)
    partial = _attention_partial(x2, Wq, K2, V2, Wo)
    return _ring_allreduce(partial)


# device time: 85488 ns/iter; 1.6293x vs baseline; 1.6293x over previous
import jax
import jax.numpy as jnp
from jax import lax
from jax.experimental import pallas as pl
from jax.experimental.pallas import tpu as pltpu

N_DEV = 16
SQ = 256
D = 1024
HQ_PER = 8
DH = 128
SKV = 4096
SCALE = 0.08838834764831843

CH = SQ // N_DEV


def _attention_partial(x2, Wq, K2, V2, Wo):

    def body(x_ref, wq_ref, k_ref, v_ref, wo_ref, out_ref):
        h = pl.program_id(0)
        q = jnp.dot(x_ref[...], wq_ref[...], preferred_element_type=jnp.float32)
        s = lax.dot_general(
            q, k_ref[...], (((1,), (1,)), ((), ())),
            preferred_element_type=jnp.float32,
        ) * SCALE
        m = jnp.max(s, axis=1, keepdims=True)
        p = jnp.exp(s - m)
        l = jnp.sum(p, axis=1, keepdims=True)
        attn = jnp.dot(p, v_ref[...], preferred_element_type=jnp.float32) / l
        contrib = jnp.dot(attn, wo_ref[...], preferred_element_type=jnp.float32)

        @pl.when(h == 0)
        def _():
            out_ref[...] = contrib

        @pl.when(h > 0)
        def _():
            out_ref[...] += contrib

    return pl.pallas_call(
        body,
        grid=(HQ_PER,),
        in_specs=[
            pl.BlockSpec((SQ, D), lambda h: (0, 0)),
            pl.BlockSpec((D, DH), lambda h: (0, h)),
            pl.BlockSpec((SKV, DH), lambda h: (0, h)),
            pl.BlockSpec((SKV, DH), lambda h: (0, h)),
            pl.BlockSpec((DH, D), lambda h: (h, 0)),
        ],
        out_specs=pl.BlockSpec((SQ, D), lambda h: (0, 0)),
        out_shape=jax.ShapeDtypeStruct((SQ, D), jnp.float32),
    )(x2, Wq, K2, V2, Wo)


def _alltoall_allreduce(partial):

    def body(p_ref, out_ref, r_ref, rs_send, rs_recv, ag_send, ag_recv):
        me = lax.axis_index("i")

        barrier_sem = pltpu.get_barrier_semaphore()
        for d in range(1, N_DEV):
            pl.semaphore_signal(
                barrier_sem, inc=1,
                device_id=(lax.rem(me + d, N_DEV),),
                device_id_type=pl.DeviceIdType.MESH,
            )
        pl.semaphore_wait(barrier_sem, N_DEV - 1)

        out_ref[0] = p_ref[...]

        rs_rdmas = []
        for d in range(1, N_DEV):
            t = lax.rem(me + d, N_DEV)
            rdma = pltpu.make_async_remote_copy(
                src_ref=out_ref.at[0, pl.ds(t * CH, CH), :],
                dst_ref=r_ref.at[N_DEV - d],
                send_sem=rs_send.at[d],
                recv_sem=rs_recv.at[N_DEV - d],
                device_id=(t,),
                device_id_type=pl.DeviceIdType.MESH,
            )
            rdma.start()
            rs_rdmas.append(rdma)
        for k in range(1, N_DEV):
            recv = pltpu.make_async_remote_copy(
                src_ref=r_ref.at[k],
                dst_ref=r_ref.at[k],
                send_sem=rs_send.at[k],
                recv_sem=rs_recv.at[k],
                device_id=(me,),
                device_id_type=pl.DeviceIdType.MESH,
            )
            recv.wait_recv()
        mine = pl.ds(me * CH, CH)
        out_ref[0, mine, :] += jnp.sum(r_ref[1:N_DEV], axis=0)

        ag_rdmas = []
        for d in range(1, N_DEV):
            t = lax.rem(me + d, N_DEV)
            rdma = pltpu.make_async_remote_copy(
                src_ref=out_ref.at[0, mine, :],
                dst_ref=out_ref.at[0, mine, :],
                send_sem=ag_send.at[d],
                recv_sem=ag_recv.at[N_DEV - d],
                device_id=(t,),
                device_id_type=pl.DeviceIdType.MESH,
            )
            rdma.start()
            ag_rdmas.append(rdma)
        for k in range(1, N_DEV):
            rows = pl.ds(lax.rem(me + k, N_DEV) * CH, CH)
            recv = pltpu.make_async_remote_copy(
                src_ref=out_ref.at[0, rows, :],
                dst_ref=out_ref.at[0, rows, :],
                send_sem=ag_send.at[k],
                recv_sem=ag_recv.at[k],
                device_id=(me,),
                device_id_type=pl.DeviceIdType.MESH,
            )
            recv.wait_recv()

        for rdma in rs_rdmas:
            rdma.wait_send()
        for rdma in ag_rdmas:
            rdma.wait_send()

    return pl.pallas_call(
        body,
        out_shape=jax.ShapeDtypeStruct((1, SQ, D), jnp.float32),
        in_specs=[pl.BlockSpec(memory_space=pltpu.VMEM)],
        out_specs=pl.BlockSpec(memory_space=pltpu.VMEM),
        scratch_shapes=[
            pltpu.VMEM((N_DEV, CH, D), jnp.float32),
            pltpu.SemaphoreType.DMA((N_DEV,)),
            pltpu.SemaphoreType.DMA((N_DEV,)),
            pltpu.SemaphoreType.DMA((N_DEV,)),
            pltpu.SemaphoreType.DMA((N_DEV,)),
        ],
        compiler_params=pltpu.CompilerParams(collective_id=0),
    )(partial)


def kernel(x, Wq, Wo, K_ext, V_ext):
    x2 = x.reshape(SQ, D)
    K2 = K_ext.reshape(SKV, HQ_PER * DH)
    V2 = V_ext.reshape(SKV, HQ_PER * DH)
    partial = _attention_partial(x2, Wq, K2, V2, Wo)
    return _alltoall_allreduce(partial)


# device time: 64543 ns/iter; 2.1581x vs baseline; 1.3245x over previous
import jax
import jax.numpy as jnp
from jax import lax
from jax.experimental import pallas as pl
from jax.experimental.pallas import tpu as pltpu

N_DEV = 16
SQ = 256
D = 1024
HQ_PER = 8
DH = 128
SKV = 4096
SCALE = 0.08838834764831843

CH = SQ // N_DEV


def _fused(x2, Wq, K4, V4, Wo):
    def body(x_ref, wq_ref, k_hbm, v_hbm, wo_ref, out_ref,
             k_buf, v_buf, copy_sems, r_ref, rs_send, rs_recv, ag_send, ag_recv):
        me = lax.axis_index("i")

        barrier_sem = pltpu.get_barrier_semaphore()
        for d in range(1, N_DEV):
            pl.semaphore_signal(
                barrier_sem, inc=1,
                device_id=(lax.rem(me + d, N_DEV),),
                device_id_type=pl.DeviceIdType.MESH,
            )

        def kv_copies(h):
            slot = h % 2
            ck = pltpu.make_async_copy(
                k_hbm.at[0, :, h, :], k_buf.at[slot], copy_sems.at[slot, 0]
            )
            cv = pltpu.make_async_copy(
                v_hbm.at[0, :, h, :], v_buf.at[slot], copy_sems.at[slot, 1]
            )
            return ck, cv

        ck, cv = kv_copies(0)
        ck.start()
        cv.start()
        for h in range(HQ_PER):
            slot = h % 2
            if h + 1 < HQ_PER:
                nk, nv = kv_copies(h + 1)
                nk.start()
                nv.start()
            q = jnp.dot(
                x_ref[...], wq_ref[:, h * DH:(h + 1) * DH],
                preferred_element_type=jnp.float32,
            )
            ck.wait()
            s = lax.dot_general(
                q, k_buf[slot], (((1,), (1,)), ((), ())),
                preferred_element_type=jnp.float32,
            ) * SCALE
            m = jnp.max(s, axis=1, keepdims=True)
            p = jnp.exp(s - m)
            l = jnp.sum(p, axis=1, keepdims=True)
            cv.wait()
            attn = jnp.dot(p, v_buf[slot], preferred_element_type=jnp.float32) / l
            contrib = jnp.dot(
                attn, wo_ref[h * DH:(h + 1) * DH, :],
                preferred_element_type=jnp.float32,
            )
            if h == 0:
                out_ref[0] = contrib
            else:
                out_ref[0] += contrib
            if h + 1 < HQ_PER:
                ck, cv = nk, nv

        pl.semaphore_wait(barrier_sem, N_DEV - 1)

        rs_rdmas = []
        for d in range(1, N_DEV):
            t = lax.rem(me + d, N_DEV)
            rdma = pltpu.make_async_remote_copy(
                src_ref=out_ref.at[0, pl.ds(t * CH, CH), :],
                dst_ref=r_ref.at[N_DEV - d],
                send_sem=rs_send.at[d],
                recv_sem=rs_recv.at[N_DEV - d],
                device_id=(t,),
                device_id_type=pl.DeviceIdType.MESH,
            )
            rdma.start()
            rs_rdmas.append(rdma)
        for k in range(1, N_DEV):
            recv = pltpu.make_async_remote_copy(
                src_ref=r_ref.at[k],
                dst_ref=r_ref.at[k],
                send_sem=rs_send.at[k],
                recv_sem=rs_recv.at[k],
                device_id=(me,),
                device_id_type=pl.DeviceIdType.MESH,
            )
            recv.wait_recv()
        mine = pl.ds(me * CH, CH)
        out_ref[0, mine, :] += jnp.sum(r_ref[1:N_DEV], axis=0)

        ag_rdmas = []
        for d in range(1, N_DEV):
            t = lax.rem(me + d, N_DEV)
            rdma = pltpu.make_async_remote_copy(
                src_ref=out_ref.at[0, mine, :],
                dst_ref=out_ref.at[0, mine, :],
                send_sem=ag_send.at[d],
                recv_sem=ag_recv.at[N_DEV - d],
                device_id=(t,),
                device_id_type=pl.DeviceIdType.MESH,
            )
            rdma.start()
            ag_rdmas.append(rdma)
        for k in range(1, N_DEV):
            rows = pl.ds(lax.rem(me + k, N_DEV) * CH, CH)
            recv = pltpu.make_async_remote_copy(
                src_ref=out_ref.at[0, rows, :],
                dst_ref=out_ref.at[0, rows, :],
                send_sem=ag_send.at[k],
                recv_sem=ag_recv.at[k],
                device_id=(me,),
                device_id_type=pl.DeviceIdType.MESH,
            )
            recv.wait_recv()

        for rdma in rs_rdmas:
            rdma.wait_send()
        for rdma in ag_rdmas:
            rdma.wait_send()

    return pl.pallas_call(
        body,
        out_shape=jax.ShapeDtypeStruct((1, SQ, D), jnp.float32),
        in_specs=[
            pl.BlockSpec(memory_space=pltpu.VMEM),
            pl.BlockSpec(memory_space=pltpu.VMEM),
            pl.BlockSpec(memory_space=pl.ANY),
            pl.BlockSpec(memory_space=pl.ANY),
            pl.BlockSpec(memory_space=pltpu.VMEM),
        ],
        out_specs=pl.BlockSpec(memory_space=pltpu.VMEM),
        scratch_shapes=[
            pltpu.VMEM((2, SKV, DH), jnp.float32),
            pltpu.VMEM((2, SKV, DH), jnp.float32),
            pltpu.SemaphoreType.DMA((2, 2)),
            pltpu.VMEM((N_DEV, CH, D), jnp.float32),
            pltpu.SemaphoreType.DMA((N_DEV,)),
            pltpu.SemaphoreType.DMA((N_DEV,)),
            pltpu.SemaphoreType.DMA((N_DEV,)),
            pltpu.SemaphoreType.DMA((N_DEV,)),
        ],
        compiler_params=pltpu.CompilerParams(collective_id=0),
    )(x2, Wq, K4, V4, Wo)


def kernel(x, Wq, Wo, K_ext, V_ext):
    x2 = x.reshape(SQ, D)
    return _fused(x2, Wq, K_ext, V_ext, Wo)


# device time: 56392 ns/iter; 2.4700x vs baseline; 1.1445x over previous
import jax
import jax.numpy as jnp
from jax import lax
from jax.experimental import pallas as pl
from jax.experimental.pallas import tpu as pltpu

N_DEV = 16
SQ = 256
D = 1024
HQ_PER = 8
DH = 128
SKV = 4096
SCALE = 0.08838834764831843

CH = SQ // N_DEV


def _fused(x2, Wq, K4, V4, Wo):
    def body(x_ref, wq_ref, k_hbm, v_hbm, wo_ref, out_ref,
             k_buf, v_buf, copy_sems, r_ref, rs_send, rs_recv, ag_send, ag_recv):
        me = lax.axis_index("i")

        barrier_sem = pltpu.get_barrier_semaphore()
        for d in range(1, N_DEV):
            pl.semaphore_signal(
                barrier_sem, inc=1,
                device_id=(lax.rem(me + d, N_DEV),),
                device_id_type=pl.DeviceIdType.MESH,
            )

        def kv_copies(h):
            slot = h % 2
            ck = pltpu.make_async_copy(
                k_hbm.at[0, :, h, :], k_buf.at[slot], copy_sems.at[slot, 0]
            )
            cv = pltpu.make_async_copy(
                v_hbm.at[0, :, h, :], v_buf.at[slot], copy_sems.at[slot, 1]
            )
            return ck, cv

        ck, cv = kv_copies(0)
        ck.start()
        cv.start()
        xb = x_ref[...].astype(jnp.bfloat16)
        for h in range(HQ_PER):
            slot = h % 2
            if h + 1 < HQ_PER:
                nk, nv = kv_copies(h + 1)
                nk.start()
                nv.start()
            q = jnp.dot(
                xb, wq_ref[:, h * DH:(h + 1) * DH].astype(jnp.bfloat16),
                preferred_element_type=jnp.float32,
            )
            ck.wait()
            s = lax.dot_general(
                q.astype(jnp.bfloat16),
                k_buf[slot].astype(jnp.bfloat16),
                (((1,), (1,)), ((), ())),
                preferred_element_type=jnp.float32,
            ) * SCALE
            m = jnp.max(s, axis=1, keepdims=True)
            p = jnp.exp(s - m)
            l = jnp.sum(p, axis=1, keepdims=True)
            cv.wait()
            attn = jnp.dot(
                p.astype(jnp.bfloat16),
                v_buf[slot].astype(jnp.bfloat16),
                preferred_element_type=jnp.float32,
            ) / l
            contrib = jnp.dot(
                attn.astype(jnp.bfloat16),
                wo_ref[h * DH:(h + 1) * DH, :].astype(jnp.bfloat16),
                preferred_element_type=jnp.float32,
            )
            if h == 0:
                out_ref[0] = contrib
            else:
                out_ref[0] += contrib
            if h + 1 < HQ_PER:
                ck, cv = nk, nv

        pl.semaphore_wait(barrier_sem, N_DEV - 1)

        rs_rdmas = []
        for d in range(1, N_DEV):
            t = lax.rem(me + d, N_DEV)
            rdma = pltpu.make_async_remote_copy(
                src_ref=out_ref.at[0, pl.ds(t * CH, CH), :],
                dst_ref=r_ref.at[N_DEV - d],
                send_sem=rs_send.at[d],
                recv_sem=rs_recv.at[N_DEV - d],
                device_id=(t,),
                device_id_type=pl.DeviceIdType.MESH,
            )
            rdma.start()
            rs_rdmas.append(rdma)
        for k in range(1, N_DEV):
            recv = pltpu.make_async_remote_copy(
                src_ref=r_ref.at[k],
                dst_ref=r_ref.at[k],
                send_sem=rs_send.at[k],
                recv_sem=rs_recv.at[k],
                device_id=(me,),
                device_id_type=pl.DeviceIdType.MESH,
            )
            recv.wait_recv()
        mine = pl.ds(me * CH, CH)
        out_ref[0, mine, :] += jnp.sum(r_ref[1:N_DEV], axis=0)

        ag_rdmas = []
        for d in range(1, N_DEV):
            t = lax.rem(me + d, N_DEV)
            rdma = pltpu.make_async_remote_copy(
                src_ref=out_ref.at[0, mine, :],
                dst_ref=out_ref.at[0, mine, :],
                send_sem=ag_send.at[d],
                recv_sem=ag_recv.at[N_DEV - d],
                device_id=(t,),
                device_id_type=pl.DeviceIdType.MESH,
            )
            rdma.start()
            ag_rdmas.append(rdma)
        for k in range(1, N_DEV):
            rows = pl.ds(lax.rem(me + k, N_DEV) * CH, CH)
            recv = pltpu.make_async_remote_copy(
                src_ref=out_ref.at[0, rows, :],
                dst_ref=out_ref.at[0, rows, :],
                send_sem=ag_send.at[k],
                recv_sem=ag_recv.at[k],
                device_id=(me,),
                device_id_type=pl.DeviceIdType.MESH,
            )
            recv.wait_recv()

        for rdma in rs_rdmas:
            rdma.wait_send()
        for rdma in ag_rdmas:
            rdma.wait_send()

    return pl.pallas_call(
        body,
        out_shape=jax.ShapeDtypeStruct((1, SQ, D), jnp.float32),
        in_specs=[
            pl.BlockSpec(memory_space=pltpu.VMEM),
            pl.BlockSpec(memory_space=pltpu.VMEM),
            pl.BlockSpec(memory_space=pl.ANY),
            pl.BlockSpec(memory_space=pl.ANY),
            pl.BlockSpec(memory_space=pltpu.VMEM),
        ],
        out_specs=pl.BlockSpec(memory_space=pltpu.VMEM),
        scratch_shapes=[
            pltpu.VMEM((2, SKV, DH), jnp.float32),
            pltpu.VMEM((2, SKV, DH), jnp.float32),
            pltpu.SemaphoreType.DMA((2, 2)),
            pltpu.VMEM((N_DEV, CH, D), jnp.float32),
            pltpu.SemaphoreType.DMA((N_DEV,)),
            pltpu.SemaphoreType.DMA((N_DEV,)),
            pltpu.SemaphoreType.DMA((N_DEV,)),
            pltpu.SemaphoreType.DMA((N_DEV,)),
        ],
        compiler_params=pltpu.CompilerParams(collective_id=0),
    )(x2, Wq, K4, V4, Wo)


def kernel(x, Wq, Wo, K_ext, V_ext):
    x2 = x.reshape(SQ, D)
    return _fused(x2, Wq, K_ext, V_ext, Wo)


# device time: 45023 ns/iter; 3.0937x vs baseline; 1.2525x over previous
import jax
import jax.numpy as jnp
from jax import lax
from jax.experimental import pallas as pl
from jax.experimental.pallas import tpu as pltpu

N_DEV = 16
SQ = 256
D = 1024
HQ_PER = 8
DH = 128
SKV = 4096
SCALE = 0.08838834764831843

CH = SQ // N_DEV


def _fused(x2, Wq, K4, V4, Wo):
    def body(x_ref, wq_ref, k_hbm, v_hbm, wo_ref, out_ref,
             k_buf, v_buf, copy_sems, pbuf, r_ref, g_ref,
             rs_send, rs_recv, ag_send, ag_recv):
        me = lax.axis_index("i")

        barrier_sem = pltpu.get_barrier_semaphore()
        for d in range(1, N_DEV):
            pl.semaphore_signal(
                barrier_sem, inc=1,
                device_id=(lax.rem(me + d, N_DEV),),
                device_id_type=pl.DeviceIdType.MESH,
            )

        def kv_copies(h):
            slot = h % 2
            ck = pltpu.make_async_copy(
                k_hbm.at[0, :, h, :], k_buf.at[slot], copy_sems.at[slot, 0]
            )
            cv = pltpu.make_async_copy(
                v_hbm.at[0, :, h, :], v_buf.at[slot], copy_sems.at[slot, 1]
            )
            return ck, cv

        ck, cv = kv_copies(0)
        ck.start()
        cv.start()
        xb = x_ref[...].astype(jnp.bfloat16)
        wqb = wq_ref[...].astype(jnp.bfloat16)
        wob = wo_ref[...].astype(jnp.bfloat16)
        qb = (
            jnp.dot(xb, wqb, preferred_element_type=jnp.float32) * SCALE
        ).astype(jnp.bfloat16)
        ones8 = jnp.ones((SKV, 8), jnp.bfloat16)
        for h in range(HQ_PER):
            slot = h % 2
            if h + 1 < HQ_PER:
                nk, nv = kv_copies(h + 1)
                nk.start()
                nv.start()
            ck.wait()
            s = lax.dot_general(
                qb[:, h * DH:(h + 1) * DH],
                k_buf[slot].astype(jnp.bfloat16),
                (((1,), (1,)), ((), ())),
                preferred_element_type=jnp.float32,
            )
            p = jnp.exp(s).astype(jnp.bfloat16)
            cv.wait()
            attn_un = jnp.dot(
                p, v_buf[slot].astype(jnp.bfloat16),
                preferred_element_type=jnp.float32,
            )
            l = jnp.dot(p, ones8, preferred_element_type=jnp.float32)
            attn = attn_un / l[:, 0:1]
            contrib = jnp.dot(
                attn.astype(jnp.bfloat16),
                wob[h * DH:(h + 1) * DH, :],
                preferred_element_type=jnp.float32,
            )
            if h == 0:
                out_ref[0] = contrib
            else:
                out_ref[0] += contrib
            if h + 1 < HQ_PER:
                ck, cv = nk, nv

        pbuf[...] = out_ref[0].astype(jnp.bfloat16)

        pl.semaphore_wait(barrier_sem, N_DEV - 1)

        rs_rdmas = []
        for d in range(1, N_DEV):
            t = lax.rem(me + d, N_DEV)
            rdma = pltpu.make_async_remote_copy(
                src_ref=pbuf.at[pl.ds(t * CH, CH), :],
                dst_ref=r_ref.at[N_DEV - d],
                send_sem=rs_send.at[d],
                recv_sem=rs_recv.at[N_DEV - d],
                device_id=(t,),
                device_id_type=pl.DeviceIdType.MESH,
            )
            rdma.start()
            rs_rdmas.append(rdma)
        for k in range(1, N_DEV):
            recv = pltpu.make_async_remote_copy(
                src_ref=r_ref.at[k],
                dst_ref=r_ref.at[k],
                send_sem=rs_send.at[k],
                recv_sem=rs_recv.at[k],
                device_id=(me,),
                device_id_type=pl.DeviceIdType.MESH,
            )
            recv.wait_recv()
        mine = pl.ds(me * CH, CH)
        out_ref[0, mine, :] += jnp.sum(
            r_ref[1:N_DEV].astype(jnp.float32), axis=0
        )
        g_ref[0] = out_ref[0, mine, :].astype(jnp.bfloat16)

        ag_rdmas = []
        for d in range(1, N_DEV):
            t = lax.rem(me + d, N_DEV)
            rdma = pltpu.make_async_remote_copy(
                src_ref=g_ref.at[0],
                dst_ref=g_ref.at[N_DEV - d],
                send_sem=ag_send.at[d],
                recv_sem=ag_recv.at[N_DEV - d],
                device_id=(t,),
                device_id_type=pl.DeviceIdType.MESH,
            )
            rdma.start()
            ag_rdmas.append(rdma)
        for k in range(1, N_DEV):
            recv = pltpu.make_async_remote_copy(
                src_ref=g_ref.at[k],
                dst_ref=g_ref.at[k],
                send_sem=ag_send.at[k],
                recv_sem=ag_recv.at[k],
                device_id=(me,),
                device_id_type=pl.DeviceIdType.MESH,
            )
            recv.wait_recv()
            rows = pl.ds(lax.rem(me + k, N_DEV) * CH, CH)
            out_ref[0, rows, :] = g_ref[k].astype(jnp.float32)

        for rdma in rs_rdmas:
            rdma.wait_send()
        for rdma in ag_rdmas:
            rdma.wait_send()

    return pl.pallas_call(
        body,
        out_shape=jax.ShapeDtypeStruct((1, SQ, D), jnp.float32),
        in_specs=[
            pl.BlockSpec(memory_space=pltpu.VMEM),
            pl.BlockSpec(memory_space=pltpu.VMEM),
            pl.BlockSpec(memory_space=pl.ANY),
            pl.BlockSpec(memory_space=pl.ANY),
            pl.BlockSpec(memory_space=pltpu.VMEM),
        ],
        out_specs=pl.BlockSpec(memory_space=pltpu.VMEM),
        scratch_shapes=[
            pltpu.VMEM((2, SKV, DH), jnp.float32),
            pltpu.VMEM((2, SKV, DH), jnp.float32),
            pltpu.SemaphoreType.DMA((2, 2)),
            pltpu.VMEM((SQ, D), jnp.bfloat16),
            pltpu.VMEM((N_DEV, CH, D), jnp.bfloat16),
            pltpu.VMEM((N_DEV, CH, D), jnp.bfloat16),
            pltpu.SemaphoreType.DMA((N_DEV,)),
            pltpu.SemaphoreType.DMA((N_DEV,)),
            pltpu.SemaphoreType.DMA((N_DEV,)),
            pltpu.SemaphoreType.DMA((N_DEV,)),
        ],
        compiler_params=pltpu.CompilerParams(collective_id=0),
    )(x2, Wq, K4, V4, Wo)


def kernel(x, Wq, Wo, K_ext, V_ext):
    x2 = x.reshape(SQ, D)
    return _fused(x2, Wq, K_ext, V_ext, Wo)
